# Initial kernel scaffold; baseline (speedup 1.0000x reference)
import jax
import jax.numpy as jnp
from jax.experimental import pallas as pl


def kernel(tokens, actions, tok_embed0, tok_embed1, tok_embed2, action_embed, level_embed, pos_embed):
    raise NotImplementedError("write your pallas kernel here")


# speedup vs baseline: 2.3906x; 2.3906x over previous
import functools

import jax
import jax.numpy as jnp
from jax import lax
from jax.experimental import pallas as pl
from jax.experimental.pallas import tpu as pltpu
from jax.experimental.pallas import tpu_sc as plsc

D = 768
LANES = 16
VECS = D // LANES
NW = 32
CHUNK = 64


def _sc_body(gidx_hbm, ct_hbm, pe_hbm, out_hbm, idx_v, rows_v, acc_v, sem):
    n_rows = out_hbm.shape[0]
    p_per_b = pe_hbm.shape[0]
    rows_per_w = n_rows // NW
    nchunks = rows_per_w // CHUNK
    wid = lax.axis_index("s") * 2 + lax.axis_index("c")

    def chunk_body(c, carry):
        base = wid * rows_per_w + c * CHUNK
        p0 = lax.rem(base, p_per_b)
        pltpu.sync_copy(gidx_hbm.at[pl.ds(base, CHUNK)], idx_v)
        gather = pltpu.async_copy(ct_hbm.at[idx_v], rows_v, sem)
        pltpu.sync_copy(pe_hbm.at[pl.ds(p0, CHUNK)], acc_v)
        gather.wait()

        def row_body(r, carry2):
            for j in range(VECS):
                sl = pl.ds(j * LANES, LANES)
                plsc.addupdate(acc_v.at[r, sl], rows_v[r, sl])
            return carry2

        lax.fori_loop(0, CHUNK, row_body, 0, unroll=False)
        pltpu.sync_copy(acc_v, out_hbm.at[pl.ds(base, CHUNK)])
        return carry

    lax.fori_loop(0, nchunks, chunk_body, 0, unroll=False)


@jax.jit
def _embed(gidx, ct, pe):
    n_rows = gidx.shape[0]
    mesh = plsc.VectorSubcoreMesh(core_axis_name="c", subcore_axis_name="s")
    f = functools.partial(
        pl.kernel,
        out_type=jax.ShapeDtypeStruct((n_rows, D), jnp.float32),
        mesh=mesh,
        scratch_types=[
            pltpu.VMEM((CHUNK,), jnp.int32),
            pltpu.VMEM((CHUNK, D), jnp.float32),
            pltpu.VMEM((CHUNK, D), jnp.float32),
            pltpu.SemaphoreType.DMA,
        ],
    )(_sc_body)
    return f(gidx, ct, pe)


def kernel(tokens, actions, tok_embed0, tok_embed1, tok_embed2, action_embed,
           level_embed, pos_embed):
    B, T, _ = tokens.shape
    num_codes = tok_embed0.shape[0]
    ct = jnp.concatenate(
        [
            tok_embed0 + level_embed[0],
            tok_embed1 + level_embed[1],
            tok_embed2 + level_embed[2],
            action_embed + level_embed[3],
        ],
        axis=0,
    )
    pe = pos_embed[: T * 4]
    gidx = jnp.stack(
        [
            tokens[..., 0],
            tokens[..., 1] + num_codes,
            tokens[..., 2] + 2 * num_codes,
            actions + 3 * num_codes,
        ],
        axis=-1,
    ).reshape(-1)
    out = _embed(gidx, ct, pe)
    return out.reshape(B, T * 4, D)


# speedup vs baseline: 2.9634x; 1.2396x over previous
import functools

import jax
import jax.numpy as jnp
from jax import lax
from jax.experimental import pallas as pl
from jax.experimental.pallas import tpu as pltpu
from jax.experimental.pallas import tpu_sc as plsc

D = 768
LANES = 16
VECS = D // LANES
NW = 32
B_PER_W = 4
CHUNK = 32
NBUF = 4


def _sc_body(gidx_hbm, ct_hbm, pe_hbm, out_hbm, idx_all, pe_v, rows, gsem, wsem):
    n_rows = out_hbm.shape[0]
    p_per_b = pe_hbm.shape[0]
    rows_per_w = n_rows // NW
    pcs = p_per_b // CHUNK
    nsteps = pcs * B_PER_W
    wid = lax.axis_index("s") * 2 + lax.axis_index("c")
    w0 = wid * rows_per_w

    def base_of(s):
        pc = s // B_PER_W
        bi = lax.rem(s, B_PER_W)
        return w0 + bi * p_per_b + pc * CHUNK, pc

    def idx_off(s):
        pc = s // B_PER_W
        bi = lax.rem(s, B_PER_W)
        return bi * p_per_b + pc * CHUNK

    pltpu.sync_copy(gidx_hbm.at[pl.ds(w0, rows_per_w)], idx_all)
    pltpu.async_copy(ct_hbm.at[idx_all.at[pl.ds(idx_off(0), CHUNK)]], rows[0], gsem[0])
    pltpu.async_copy(ct_hbm.at[idx_all.at[pl.ds(idx_off(1), CHUNK)]], rows[1], gsem[1])

    def outer(i, carry):
        for k in range(NBUF):
            s = i * NBUF + k
            base, pc = base_of(s)
            rx, gs, ws = rows[k], gsem[k], wsem[k]
            k2 = (k + 2) % NBUF
            @pl.when(s >= 2)
            def _():
                pltpu.make_async_copy(rows[k2], out_hbm.at[pl.ds(base, CHUNK)],
                                      wsem[k2]).wait()

            @pl.when(s + 2 < nsteps)
            def _():
                pltpu.async_copy(
                    ct_hbm.at[idx_all.at[pl.ds(idx_off(s + 2), CHUNK)]],
                    rows[k2], gsem[k2])

            if k == 0:
                pltpu.sync_copy(pe_hbm.at[pl.ds(pc * CHUNK, CHUNK)], pe_v)

            pltpu.make_async_copy(ct_hbm.at[idx_all.at[pl.ds(0, CHUNK)]],
                                  rx, gs).wait()

            def row_body(r, carry2):
                for j in range(VECS):
                    sl = pl.ds(j * LANES, LANES)
                    plsc.addupdate(rx.at[r, sl], pe_v[r, sl])
                return carry2

            lax.fori_loop(0, CHUNK, row_body, 0, unroll=False)
            pltpu.async_copy(rx, out_hbm.at[pl.ds(base, CHUNK)], ws)
        return carry

    lax.fori_loop(0, nsteps // NBUF, outer, 0, unroll=False)
    for k in ((nsteps - 2) % NBUF, (nsteps - 1) % NBUF):
        pltpu.make_async_copy(rows[k], out_hbm.at[pl.ds(w0, CHUNK)],
                              wsem[k]).wait()


@jax.jit
def _embed(gidx, ct, pe):
    n_rows = gidx.shape[0]
    mesh = plsc.VectorSubcoreMesh(core_axis_name="c", subcore_axis_name="s")
    f = functools.partial(
        pl.kernel,
        out_type=jax.ShapeDtypeStruct((n_rows, D), jnp.float32),
        mesh=mesh,
        scratch_types=[
            pltpu.VMEM((n_rows // NW,), jnp.int32),
            pltpu.VMEM((CHUNK, D), jnp.float32),
            [pltpu.VMEM((CHUNK, D), jnp.float32)] * NBUF,
            [pltpu.SemaphoreType.DMA] * NBUF,
            [pltpu.SemaphoreType.DMA] * NBUF,
        ],
    )(_sc_body)
    return f(gidx, ct, pe)


def kernel(tokens, actions, tok_embed0, tok_embed1, tok_embed2, action_embed,
           level_embed, pos_embed):
    B, T, _ = tokens.shape
    num_codes = tok_embed0.shape[0]
    ct = jnp.concatenate(
        [
            tok_embed0 + level_embed[0],
            tok_embed1 + level_embed[1],
            tok_embed2 + level_embed[2],
            action_embed + level_embed[3],
        ],
        axis=0,
    )
    pe = pos_embed[: T * 4]
    gidx = jnp.stack(
        [
            tokens[..., 0],
            tokens[..., 1] + num_codes,
            tokens[..., 2] + 2 * num_codes,
            actions + 3 * num_codes,
        ],
        axis=-1,
    ).reshape(-1)
    out = _embed(gidx, ct, pe)
    return out.reshape(B, T * 4, D)
